# baseline (speedup 1.0000x reference)
import functools

import jax
import jax.numpy as jnp
from jax import lax
from jax.experimental import pallas as pl
from jax.experimental.pallas import tpu as pltpu
from jax.experimental.pallas import tpu_sc as plsc

N = 10000
E = 320000
F_IN = 128
H = 64
C = 10

ND = 10240
NW = 32
E_W = E // NW
CH = 128
NF = E_W // CH
TAIL = E_W - NF * CH
TRD = ND // 16
TR = ND // 16
GRP = 6
ZR = 64
RB = N // 5

_MESH = plsc.VectorSubcoreMesh(core_axis_name="c", subcore_axis_name="s")
_SC_PARAMS = pltpu.CompilerParams(use_tc_tiling_on_sc=False)


def _deg_kernel(dstm_hbm, dstt_hbm, out_hbm, idx_v, idxt_v, ones_v, zer_v,
                acc_sh):
    c = lax.axis_index("c")
    s = lax.axis_index("s")
    wid = s * 2 + c

    def fill(i, carry):
        ones_v[pl.ds(i * 16, 16)] = jnp.full((16,), 1.0, jnp.float32)
        return carry

    lax.fori_loop(0, CH // 16, fill, 0)

    def fillz(i, carry):
        zer_v[pl.ds(i * 16, 16)] = jnp.zeros((16,), jnp.float32)
        return carry

    lax.fori_loop(0, TRD // 16, fillz, 0)
    pltpu.sync_copy(zer_v, acc_sh.at[pl.ds(s * TRD, TRD)])
    plsc.subcore_barrier()

    pltpu.sync_copy(dstm_hbm.at[wid], idx_v)
    pltpu.sync_copy(dstt_hbm.at[wid], idxt_v)

    def body(j, carry):
        pltpu.sync_copy(ones_v, acc_sh.at[idx_v.at[j]], add=True)
        return carry

    lax.fori_loop(0, NF, body, 0)
    pltpu.sync_copy(ones_v.at[pl.ds(0, TAIL)], acc_sh.at[idxt_v.at[0]],
                    add=True)
    plsc.subcore_barrier()
    pltpu.sync_copy(acc_sh.at[pl.ds(s * TRD, TRD)],
                    out_hbm.at[c, pl.ds(s * TRD, TRD)])


_deg_call = functools.partial(
    pl.kernel,
    out_type=jax.ShapeDtypeStruct((2, ND), jnp.float32),
    mesh=_MESH,
    scratch_types=[
        pltpu.VMEM((NF, CH), jnp.int32),
        pltpu.VMEM((1, TAIL), jnp.int32),
        pltpu.VMEM((CH,), jnp.float32),
        pltpu.VMEM((TRD,), jnp.float32),
        pltpu.VMEM_SHARED((ND,), jnp.float32),
    ],
)(_deg_kernel)


def _make_seg_kernel(D):
    per_row = D // 16

    def seg_kernel(y_hbm, srcm_hbm, dstm_hbm, srct_hbm, dstt_hbm, out_hbm,
                   srcm_v, dstm_v, srct_v, dstt_v, bufs, tbuf, zer_v,
                   acc_sh, gsems, ssems, isem):
        c = lax.axis_index("c")
        s = lax.axis_index("s")
        wid = s * 2 + c

        def fillz(t, carry):
            zer_v[t // per_row, pl.ds((t % per_row) * 16, 16)] = (
                jnp.zeros((16,), jnp.float32))
            return carry

        lax.fori_loop(0, ZR * per_row, fillz, 0)

        for t in range(TR // ZR):
            pltpu.async_copy(zer_v, acc_sh.at[pl.ds(s * TR + t * ZR, ZR)],
                             isem)
        for t in range(TR // ZR):
            pltpu.make_async_copy(
                zer_v, acc_sh.at[pl.ds(s * TR + t * ZR, ZR)], isem).wait()
        pltpu.sync_copy(srcm_hbm.at[wid], srcm_v)
        pltpu.sync_copy(dstm_hbm.at[wid], dstm_v)
        pltpu.sync_copy(srct_hbm.at[wid], srct_v)
        pltpu.sync_copy(dstt_hbm.at[wid], dstt_v)
        plsc.subcore_barrier()

        for t in range(GRP):
            pltpu.async_copy(y_hbm.at[srcm_v.at[t]], bufs[t], gsems[t])

        def body(g, carry):
            j0 = g * GRP
            sdescs = []
            for t in range(GRP):
                pltpu.make_async_copy(y_hbm.at[srcm_v.at[j0 + t]], bufs[t],
                                      gsems[t]).wait()
                sdescs.append(
                    pltpu.async_copy(bufs[t], acc_sh.at[dstm_v.at[j0 + t]],
                                     ssems[t], add=True))
            for t in range(GRP):
                sdescs[t].wait()

                @pl.when(j0 + GRP + t < NF)
                def _():
                    pltpu.async_copy(y_hbm.at[srcm_v.at[j0 + GRP + t]],
                                     bufs[t], gsems[t])
            return carry

        lax.fori_loop(0, NF // GRP, body, 0)

        pltpu.sync_copy(y_hbm.at[srct_v.at[0]], tbuf)
        pltpu.sync_copy(tbuf, acc_sh.at[dstt_v.at[0]], add=True)
        plsc.subcore_barrier()

        for t in range(TR // ZR):
            pltpu.async_copy(
                acc_sh.at[pl.ds(s * TR + t * ZR, ZR)],
                out_hbm.at[c, pl.ds(s * TR + t * ZR, ZR), pl.ds(0, D)],
                isem)
        for t in range(TR // ZR):
            pltpu.make_async_copy(
                acc_sh.at[pl.ds(s * TR + t * ZR, ZR)],
                out_hbm.at[c, pl.ds(s * TR + t * ZR, ZR), pl.ds(0, D)],
                isem).wait()

    return functools.partial(
        pl.kernel,
        out_type=jax.ShapeDtypeStruct((2, ND, 128), jnp.float32),
        mesh=_MESH,
        scratch_types=[
            pltpu.VMEM((NF, CH), jnp.int32),
            pltpu.VMEM((NF, CH), jnp.int32),
            pltpu.VMEM((1, TAIL), jnp.int32),
            pltpu.VMEM((1, TAIL), jnp.int32),
            [pltpu.VMEM((CH, D), jnp.float32) for _ in range(GRP)],
            pltpu.VMEM((TAIL, D), jnp.float32),
            pltpu.VMEM((ZR, D), jnp.float32),
            pltpu.VMEM_SHARED((ND, D), jnp.float32),
            [pltpu.SemaphoreType.DMA for _ in range(GRP)],
            [pltpu.SemaphoreType.DMA for _ in range(GRP)],
            pltpu.SemaphoreType.DMA,
        ],
        compiler_params=_SC_PARAMS,
    )(seg_kernel)


_seg64_call = _make_seg_kernel(H)
_seg16_call = _make_seg_kernel(16)


def _dinv_of(degp_ref):
    d = degp_ref[:, 0:1] + degp_ref[:, 1:2] + 1.0
    return lax.rsqrt(jnp.maximum(d, 1.0))


def _tc1_body(x_ref, degp_ref, w1_ref, y_ref):
    dinv = _dinv_of(degp_ref)
    xw = jnp.dot(x_ref[...], w1_ref[...], preferred_element_type=jnp.float32)
    y_ref[...] = dinv * xw


def _tc2_body(s0_ref, s1_ref, y1_ref, degp_ref, b1_ref, w2_ref, y2_ref):
    dinv = _dinv_of(degp_ref)
    tot = s0_ref[0, :, :H] + s1_ref[0, :, :H] + y1_ref[...]
    h = jnp.maximum(dinv * tot + b1_ref[...], 0.0)
    y2_ref[...] = dinv * jnp.dot(h, w2_ref[...],
                                 preferred_element_type=jnp.float32)


def _tc3_body(s0_ref, s1_ref, y2_ref, degp_ref, b2_ref, o_ref):
    dinv = _dinv_of(degp_ref)
    o = dinv * (s0_ref[0, :, :16] + s1_ref[0, :, :16] + y2_ref[...]) + (
        b2_ref[...])
    col = lax.broadcasted_iota(jnp.int32, o.shape, 1)
    mask = col < C
    om = jnp.where(mask, o, -1e30)
    m = jnp.max(om, axis=1, keepdims=True)
    e = jnp.where(mask, jnp.exp(om - m), 0.0)
    lse = jnp.log(jnp.sum(e, axis=1, keepdims=True)) + m
    o_ref[...] = (o - lse)[:, :C]


def _row_spec(width):
    return pl.BlockSpec((RB, width), lambda i: (i, 0))


def _full_spec(r, w):
    return pl.BlockSpec((r, w), lambda i: (0, 0))


def _part_spec(core):
    return pl.BlockSpec((1, RB, 128), lambda i: (core, i, 0))


_tc1_call = pl.pallas_call(
    _tc1_body,
    grid=(N // RB,),
    in_specs=[_row_spec(F_IN), _row_spec(2), _full_spec(F_IN, H)],
    out_specs=_row_spec(H),
    out_shape=jax.ShapeDtypeStruct((N, H), jnp.float32),
)

_tc2_call = pl.pallas_call(
    _tc2_body,
    grid=(N // RB,),
    in_specs=[_part_spec(0), _part_spec(1), _row_spec(H), _row_spec(2),
              _full_spec(1, H), _full_spec(H, 16)],
    out_specs=_row_spec(16),
    out_shape=jax.ShapeDtypeStruct((N, 16), jnp.float32),
)

_tc3_call = pl.pallas_call(
    _tc3_body,
    grid=(N // RB,),
    in_specs=[_part_spec(0), _part_spec(1), _row_spec(16), _row_spec(2),
              _full_spec(1, 16)],
    out_specs=_row_spec(C),
    out_shape=jax.ShapeDtypeStruct((N, C), jnp.float32),
)


def _gcn_forward(x, edge_index, W1, b1, W2, b2):
    ei = edge_index.astype(jnp.int32)
    srcw = ei[0].reshape(NW, E_W)
    dstw = ei[1].reshape(NW, E_W)
    src_m = srcw[:, :NF * CH].reshape(NW, NF, CH)
    dst_m = dstw[:, :NF * CH].reshape(NW, NF, CH)
    src_t = srcw[:, NF * CH:].reshape(NW, 1, TAIL)
    dst_t = dstw[:, NF * CH:].reshape(NW, 1, TAIL)

    w2_pad = jnp.zeros((H, 16), jnp.float32).at[:, :C].set(W2)
    b1r = b1.reshape(1, H)
    b2_pad = jnp.zeros((1, 16), jnp.float32).at[0, :C].set(b2)

    degp = _deg_call(dst_m, dst_t)
    degp_t = degp.T

    y1 = _tc1_call(x, degp_t, W1)
    s1 = _seg64_call(y1, src_m, dst_m, src_t, dst_t)
    y2 = _tc2_call(s1, s1, y1, degp_t, b1r, w2_pad)
    s2 = _seg16_call(y2, src_m, dst_m, src_t, dst_t)
    return _tc3_call(s2, s2, y2, degp_t, b2_pad)


kernel = jax.jit(_gcn_forward)


# speedup vs baseline: 1.0244x; 1.0244x over previous
import functools

import jax
import jax.numpy as jnp
from jax import lax
from jax.experimental import pallas as pl
from jax.experimental.pallas import tpu as pltpu
from jax.experimental.pallas import tpu_sc as plsc

N = 10000
E = 320000
F_IN = 128
H = 64
C = 10

ND = 10240
NW = 32
E_W = E // NW
CH = 128
NF = E_W // CH
TAIL = E_W - NF * CH
TRD = ND // 16
TR = ND // 16
GRP = 6
ZR = 64
RB = N // 2

_MESH = plsc.VectorSubcoreMesh(core_axis_name="c", subcore_axis_name="s")
_SC_PARAMS = pltpu.CompilerParams(use_tc_tiling_on_sc=False)


def _deg_kernel(dstm_hbm, dstt_hbm, out_hbm, idx_v, idxt_v, ones_v, zer_v,
                acc_sh):
    c = lax.axis_index("c")
    s = lax.axis_index("s")
    wid = s * 2 + c

    def fill(i, carry):
        ones_v[pl.ds(i * 16, 16)] = jnp.full((16,), 1.0, jnp.float32)
        return carry

    lax.fori_loop(0, CH // 16, fill, 0)

    def fillz(i, carry):
        zer_v[pl.ds(i * 16, 16)] = jnp.zeros((16,), jnp.float32)
        return carry

    lax.fori_loop(0, TRD // 16, fillz, 0)
    pltpu.sync_copy(zer_v, acc_sh.at[pl.ds(s * TRD, TRD)])
    plsc.subcore_barrier()

    pltpu.sync_copy(dstm_hbm.at[wid], idx_v)
    pltpu.sync_copy(dstt_hbm.at[wid], idxt_v)

    def body(j, carry):
        pltpu.sync_copy(ones_v, acc_sh.at[idx_v.at[j]], add=True)
        return carry

    lax.fori_loop(0, NF, body, 0)
    pltpu.sync_copy(ones_v.at[pl.ds(0, TAIL)], acc_sh.at[idxt_v.at[0]],
                    add=True)
    plsc.subcore_barrier()
    pltpu.sync_copy(acc_sh.at[pl.ds(s * TRD, TRD)],
                    out_hbm.at[c, pl.ds(s * TRD, TRD)])


_deg_call = functools.partial(
    pl.kernel,
    out_type=jax.ShapeDtypeStruct((2, ND), jnp.float32),
    mesh=_MESH,
    scratch_types=[
        pltpu.VMEM((NF, CH), jnp.int32),
        pltpu.VMEM((1, TAIL), jnp.int32),
        pltpu.VMEM((CH,), jnp.float32),
        pltpu.VMEM((TRD,), jnp.float32),
        pltpu.VMEM_SHARED((ND,), jnp.float32),
    ],
)(_deg_kernel)


def _make_seg_kernel(D):
    per_row = D // 16

    def seg_kernel(y_hbm, srcm_hbm, dstm_hbm, srct_hbm, dstt_hbm, out_hbm,
                   srcm_v, dstm_v, srct_v, dstt_v, bufs, tbuf, zer_v,
                   acc_sh, gsems, ssems, isem):
        c = lax.axis_index("c")
        s = lax.axis_index("s")
        wid = s * 2 + c

        def fillz(t, carry):
            zer_v[t // per_row, pl.ds((t % per_row) * 16, 16)] = (
                jnp.zeros((16,), jnp.float32))
            return carry

        lax.fori_loop(0, ZR * per_row, fillz, 0)

        for t in range(TR // ZR):
            pltpu.async_copy(zer_v, acc_sh.at[pl.ds(s * TR + t * ZR, ZR)],
                             isem)
        for t in range(TR // ZR):
            pltpu.make_async_copy(
                zer_v, acc_sh.at[pl.ds(s * TR + t * ZR, ZR)], isem).wait()
        pltpu.sync_copy(srcm_hbm.at[wid], srcm_v)
        pltpu.sync_copy(dstm_hbm.at[wid], dstm_v)
        pltpu.sync_copy(srct_hbm.at[wid], srct_v)
        pltpu.sync_copy(dstt_hbm.at[wid], dstt_v)
        plsc.subcore_barrier()

        for t in range(GRP):
            pltpu.async_copy(y_hbm.at[srcm_v.at[t]], bufs[t], gsems[t])

        def body(g, carry):
            j0 = g * GRP
            sdescs = []
            for t in range(GRP):
                pltpu.make_async_copy(y_hbm.at[srcm_v.at[j0 + t]], bufs[t],
                                      gsems[t]).wait()
                sdescs.append(
                    pltpu.async_copy(bufs[t], acc_sh.at[dstm_v.at[j0 + t]],
                                     ssems[t], add=True))
            for t in range(GRP):
                sdescs[t].wait()

                @pl.when(j0 + GRP + t < NF)
                def _():
                    pltpu.async_copy(y_hbm.at[srcm_v.at[j0 + GRP + t]],
                                     bufs[t], gsems[t])
            return carry

        lax.fori_loop(0, NF // GRP, body, 0)

        pltpu.sync_copy(y_hbm.at[srct_v.at[0]], tbuf)
        pltpu.sync_copy(tbuf, acc_sh.at[dstt_v.at[0]], add=True)
        plsc.subcore_barrier()

        for t in range(TR // ZR):
            pltpu.async_copy(
                acc_sh.at[pl.ds(s * TR + t * ZR, ZR)],
                out_hbm.at[c, pl.ds(s * TR + t * ZR, ZR), pl.ds(0, D)],
                isem)
        for t in range(TR // ZR):
            pltpu.make_async_copy(
                acc_sh.at[pl.ds(s * TR + t * ZR, ZR)],
                out_hbm.at[c, pl.ds(s * TR + t * ZR, ZR), pl.ds(0, D)],
                isem).wait()

    return functools.partial(
        pl.kernel,
        out_type=jax.ShapeDtypeStruct((2, ND, 128), jnp.float32),
        mesh=_MESH,
        scratch_types=[
            pltpu.VMEM((NF, CH), jnp.int32),
            pltpu.VMEM((NF, CH), jnp.int32),
            pltpu.VMEM((1, TAIL), jnp.int32),
            pltpu.VMEM((1, TAIL), jnp.int32),
            [pltpu.VMEM((CH, D), jnp.float32) for _ in range(GRP)],
            pltpu.VMEM((TAIL, D), jnp.float32),
            pltpu.VMEM((ZR, D), jnp.float32),
            pltpu.VMEM_SHARED((ND, D), jnp.float32),
            [pltpu.SemaphoreType.DMA for _ in range(GRP)],
            [pltpu.SemaphoreType.DMA for _ in range(GRP)],
            pltpu.SemaphoreType.DMA,
        ],
        compiler_params=_SC_PARAMS,
    )(seg_kernel)


_seg64_call = _make_seg_kernel(H)
_seg16_call = _make_seg_kernel(16)


def _dinv_of(degp_ref):
    d = degp_ref[:, 0:1] + degp_ref[:, 1:2] + 1.0
    return lax.rsqrt(jnp.maximum(d, 1.0))


def _tc1_body(x_ref, degp_ref, w1_ref, y_ref):
    dinv = _dinv_of(degp_ref)
    xw = jnp.dot(x_ref[...], w1_ref[...], preferred_element_type=jnp.float32)
    y_ref[...] = dinv * xw


def _tc2_body(s0_ref, s1_ref, y1_ref, degp_ref, b1_ref, w2_ref, y2_ref):
    dinv = _dinv_of(degp_ref)
    tot = s0_ref[0, :, :H] + s1_ref[0, :, :H] + y1_ref[...]
    h = jnp.maximum(dinv * tot + b1_ref[...], 0.0)
    y2_ref[...] = dinv * jnp.dot(h, w2_ref[...],
                                 preferred_element_type=jnp.float32)


def _tc3_body(s0_ref, s1_ref, y2_ref, degp_ref, b2_ref, o_ref):
    dinv = _dinv_of(degp_ref)
    o = dinv * (s0_ref[0, :, :16] + s1_ref[0, :, :16] + y2_ref[...]) + (
        b2_ref[...])
    col = lax.broadcasted_iota(jnp.int32, o.shape, 1)
    mask = col < C
    om = jnp.where(mask, o, -1e30)
    m = jnp.max(om, axis=1, keepdims=True)
    e = jnp.where(mask, jnp.exp(om - m), 0.0)
    lse = jnp.log(jnp.sum(e, axis=1, keepdims=True)) + m
    o_ref[...] = (o - lse)[:, :C]


def _row_spec(width):
    return pl.BlockSpec((RB, width), lambda i: (i, 0))


def _full_spec(r, w):
    return pl.BlockSpec((r, w), lambda i: (0, 0))


def _part_spec(core):
    return pl.BlockSpec((1, RB, 128), lambda i: (core, i, 0))


_tc1_call = pl.pallas_call(
    _tc1_body,
    grid=(N // RB,),
    in_specs=[_row_spec(F_IN), _row_spec(2), _full_spec(F_IN, H)],
    out_specs=_row_spec(H),
    out_shape=jax.ShapeDtypeStruct((N, H), jnp.float32),
)

_tc2_call = pl.pallas_call(
    _tc2_body,
    grid=(N // RB,),
    in_specs=[_part_spec(0), _part_spec(1), _row_spec(H), _row_spec(2),
              _full_spec(1, H), _full_spec(H, 16)],
    out_specs=_row_spec(16),
    out_shape=jax.ShapeDtypeStruct((N, 16), jnp.float32),
)

_tc3_call = pl.pallas_call(
    _tc3_body,
    grid=(N // RB,),
    in_specs=[_part_spec(0), _part_spec(1), _row_spec(16), _row_spec(2),
              _full_spec(1, 16)],
    out_specs=_row_spec(C),
    out_shape=jax.ShapeDtypeStruct((N, C), jnp.float32),
)


def _gcn_forward(x, edge_index, W1, b1, W2, b2):
    ei = edge_index.astype(jnp.int32)
    srcw = ei[0].reshape(NW, E_W)
    dstw = ei[1].reshape(NW, E_W)
    src_m = srcw[:, :NF * CH].reshape(NW, NF, CH)
    dst_m = dstw[:, :NF * CH].reshape(NW, NF, CH)
    src_t = srcw[:, NF * CH:].reshape(NW, 1, TAIL)
    dst_t = dstw[:, NF * CH:].reshape(NW, 1, TAIL)

    w2_pad = jnp.zeros((H, 16), jnp.float32).at[:, :C].set(W2)
    b1r = b1.reshape(1, H)
    b2_pad = jnp.zeros((1, 16), jnp.float32).at[0, :C].set(b2)

    degp = _deg_call(dst_m, dst_t)
    degp_t = degp.T

    y1 = _tc1_call(x, degp_t, W1)
    s1 = _seg64_call(y1, src_m, dst_m, src_t, dst_t)
    y2 = _tc2_call(s1, s1, y1, degp_t, b1r, w2_pad)
    s2 = _seg16_call(y2, src_m, dst_m, src_t, dst_t)
    return _tc3_call(s2, s2, y2, degp_t, b2_pad)


kernel = jax.jit(_gcn_forward)
